# Initial kernel scaffold; baseline (speedup 1.0000x reference)
import jax
import jax.numpy as jnp
from jax.experimental import pallas as pl


def kernel(probs, labels):
    raise NotImplementedError("write your pallas kernel here")


# speedup vs baseline: 1.1671x; 1.1671x over previous
import jax
import jax.numpy as jnp
from jax import lax
from jax.experimental import pallas as pl
from jax.experimental.pallas import tpu as pltpu

_N_BINS = 15
_NB = 100
_MAX_BITS = 0x3F800000


def _pass1(probs_ref, lab_ref, conf_ref, acc_ref):
    x = probs_ref[...]
    r = x.shape[0]
    col = lax.broadcasted_iota(jnp.int32, x.shape, 1)
    x = jnp.where(col == 1, -9999.0, x)
    conf = jnp.max(x, axis=1, keepdims=True)
    first = jnp.min(jnp.where(x == conf, col, x.shape[1]), axis=1, keepdims=True)
    conf_row = jnp.transpose(conf)
    first_row = jnp.transpose(first)
    lab = lab_ref[...].reshape(1, r)
    acc_row = (first_row == lab).astype(jnp.float32)
    conf_ref[...] = conf_row.reshape(1, 1, r)
    acc_ref[...] = acc_row.reshape(1, 1, r)


def _pass2(conf_ref, acc_ref, ranks_ref, frac_ref, out_ref, bits_ref):
    nb, _, r = conf_ref.shape
    n = nb * r
    bits_ref[...] = lax.bitcast_convert_type(
        conf_ref[...].reshape(nb, r), jnp.int32)

    bounds = []
    for j in range(16):
        rk = ranks_ref[j]

        def body(_, st):
            lo, hi = st
            mid = lo + (hi - lo) // 2
            c = jnp.sum((bits_ref[...] <= mid).astype(jnp.int32))
            take = c >= rk + 1
            return (jnp.where(take, lo, mid + 1), jnp.where(take, mid, hi))

        lo, _ = lax.fori_loop(0, 30, body,
                              (jnp.int32(0), jnp.int32(_MAX_BITS - 1)))
        b = bits_ref[...]
        cnt_le = jnp.sum((b <= lo).astype(jnp.int32))
        nxt = jnp.min(jnp.where(b > lo, b, jnp.int32(0x7F7FFFFF)))
        hi_bits = jnp.where(cnt_le >= rk + 2, lo, nxt)
        v_lo = lax.bitcast_convert_type(lo, jnp.float32)
        v_hi = lax.bitcast_convert_type(hi_bits, jnp.float32)
        f = frac_ref[j]
        bounds.append(v_lo + f * (v_hi - v_lo))

    ece = jnp.float32(0.0)
    conf = conf_ref[...].reshape(nb, r)
    acc = acc_ref[...].reshape(nb, r)
    for i in range(_N_BINS):
        inb = (conf > bounds[i]) & (conf <= bounds[i + 1])
        cnt = jnp.sum(jnp.where(inb, 1.0, 0.0))
        sc = jnp.sum(jnp.where(inb, conf, 0.0))
        sa = jnp.sum(jnp.where(inb, acc, 0.0))
        safe = jnp.maximum(cnt, 1.0)
        contrib = jnp.abs(sc / safe - sa / safe) * (cnt / n)
        ece = ece + jnp.where(cnt > 0, contrib, 0.0)
    out_ref[0, 0] = ece


def kernel(probs, labels):
    n, c = probs.shape
    r = n // _NB
    labels3 = labels.astype(jnp.int32).reshape(_NB, 1, r)

    conf_t, acc_t = pl.pallas_call(
        _pass1,
        grid=(_NB,),
        in_specs=[
            pl.BlockSpec((r, c), lambda i: (i, 0)),
            pl.BlockSpec((1, 1, r), lambda i: (i, 0, 0)),
        ],
        out_specs=[
            pl.BlockSpec((1, 1, r), lambda i: (i, 0, 0)),
            pl.BlockSpec((1, 1, r), lambda i: (i, 0, 0)),
        ],
        out_shape=[
            jax.ShapeDtypeStruct((_NB, 1, r), jnp.float32),
            jax.ShapeDtypeStruct((_NB, 1, r), jnp.float32),
        ],
    )(probs, labels3)

    xq = jnp.linspace(0.0, float(n), _N_BINS + 1)
    ilo = jnp.clip(jnp.floor(xq), 0, n - 1).astype(jnp.int32)
    frac = jnp.clip(xq - ilo.astype(jnp.float32), 0.0, 1.0)
    frac = jnp.where(ilo >= n - 1, 0.0, frac).astype(jnp.float32)

    ece = pl.pallas_call(
        _pass2,
        in_specs=[
            pl.BlockSpec((_NB, 1, r), lambda: (0, 0, 0)),
            pl.BlockSpec((_NB, 1, r), lambda: (0, 0, 0)),
            pl.BlockSpec(memory_space=pltpu.SMEM),
            pl.BlockSpec(memory_space=pltpu.SMEM),
        ],
        out_specs=pl.BlockSpec(memory_space=pltpu.SMEM),
        out_shape=jax.ShapeDtypeStruct((1, 1), jnp.float32),
        scratch_shapes=[pltpu.VMEM((_NB, r), jnp.int32)],
    )(conf_t, acc_t, ilo, frac)

    return ece.reshape(1)


# speedup vs baseline: 1.5111x; 1.2947x over previous
import jax
import jax.numpy as jnp
from jax import lax
from jax.experimental import pallas as pl
from jax.experimental.pallas import tpu as pltpu

_N_BINS = 15
_NB = 100
_MAX_BITS = 0x3F800000


def _pass1(probs_ref, lab_ref, conf_ref, acc_ref):
    x = probs_ref[...]
    r = x.shape[0]
    col = lax.broadcasted_iota(jnp.int32, x.shape, 1)
    x = jnp.where(col == 1, -9999.0, x)
    conf = jnp.max(x, axis=1, keepdims=True)
    first = jnp.min(jnp.where(x == conf, col, x.shape[1]), axis=1, keepdims=True)
    conf_row = jnp.transpose(conf)
    first_row = jnp.transpose(first)
    lab = lab_ref[...].reshape(1, r)
    acc_row = (first_row == lab).astype(jnp.float32)
    conf_ref[...] = conf_row.reshape(1, 1, r)
    acc_ref[...] = acc_row.reshape(1, 1, r)


def _pass2(conf_ref, acc_ref, ranks_ref, frac_ref, out_ref, bits_ref):
    nb, _, r = conf_ref.shape
    n = nb * r
    bits_ref[...] = lax.bitcast_convert_type(
        conf_ref[...].reshape(nb, r), jnp.int32)

    ranks = [ranks_ref[j] for j in range(16)]

    def body(_, st):
        los, his = st
        mids = [lo + (hi - lo) // 2 for lo, hi in zip(los, his)]
        b = bits_ref[...]
        new_lo, new_hi = [], []
        for j in range(16):
            c = jnp.sum((b <= mids[j]).astype(jnp.int32))
            take = c >= ranks[j] + 1
            new_lo.append(jnp.where(take, los[j], mids[j] + 1))
            new_hi.append(jnp.where(take, mids[j], his[j]))
        return (tuple(new_lo), tuple(new_hi))

    init = (tuple(jnp.int32(0) for _ in range(16)),
            tuple(jnp.int32(_MAX_BITS - 1) for _ in range(16)))
    los, _ = lax.fori_loop(0, 30, body, init)

    b = bits_ref[...]
    bounds = []
    for j in range(16):
        lo = los[j]
        cnt_le = jnp.sum((b <= lo).astype(jnp.int32))
        nxt = jnp.min(jnp.where(b > lo, b, jnp.int32(0x7F7FFFFF)))
        hi_bits = jnp.where(cnt_le >= ranks[j] + 2, lo, nxt)
        v_lo = lax.bitcast_convert_type(lo, jnp.float32)
        v_hi = lax.bitcast_convert_type(hi_bits, jnp.float32)
        f = frac_ref[j]
        bounds.append(v_lo + f * (v_hi - v_lo))

    ece = jnp.float32(0.0)
    conf = conf_ref[...].reshape(nb, r)
    acc = acc_ref[...].reshape(nb, r)
    for i in range(_N_BINS):
        inb = (conf > bounds[i]) & (conf <= bounds[i + 1])
        cnt = jnp.sum(jnp.where(inb, 1.0, 0.0))
        sc = jnp.sum(jnp.where(inb, conf, 0.0))
        sa = jnp.sum(jnp.where(inb, acc, 0.0))
        safe = jnp.maximum(cnt, 1.0)
        contrib = jnp.abs(sc / safe - sa / safe) * (cnt / n)
        ece = ece + jnp.where(cnt > 0, contrib, 0.0)
    out_ref[0, 0] = ece


def kernel(probs, labels):
    n, c = probs.shape
    r = n // _NB
    labels3 = labels.astype(jnp.int32).reshape(_NB, 1, r)

    conf_t, acc_t = pl.pallas_call(
        _pass1,
        grid=(_NB,),
        in_specs=[
            pl.BlockSpec((r, c), lambda i: (i, 0)),
            pl.BlockSpec((1, 1, r), lambda i: (i, 0, 0)),
        ],
        out_specs=[
            pl.BlockSpec((1, 1, r), lambda i: (i, 0, 0)),
            pl.BlockSpec((1, 1, r), lambda i: (i, 0, 0)),
        ],
        out_shape=[
            jax.ShapeDtypeStruct((_NB, 1, r), jnp.float32),
            jax.ShapeDtypeStruct((_NB, 1, r), jnp.float32),
        ],
    )(probs, labels3)

    xq = jnp.linspace(0.0, float(n), _N_BINS + 1)
    ilo = jnp.clip(jnp.floor(xq), 0, n - 1).astype(jnp.int32)
    frac = jnp.clip(xq - ilo.astype(jnp.float32), 0.0, 1.0)
    frac = jnp.where(ilo >= n - 1, 0.0, frac).astype(jnp.float32)

    ece = pl.pallas_call(
        _pass2,
        in_specs=[
            pl.BlockSpec((_NB, 1, r), lambda: (0, 0, 0)),
            pl.BlockSpec((_NB, 1, r), lambda: (0, 0, 0)),
            pl.BlockSpec(memory_space=pltpu.SMEM),
            pl.BlockSpec(memory_space=pltpu.SMEM),
        ],
        out_specs=pl.BlockSpec(memory_space=pltpu.SMEM),
        out_shape=jax.ShapeDtypeStruct((1, 1), jnp.float32),
        scratch_shapes=[pltpu.VMEM((_NB, r), jnp.int32)],
    )(conf_t, acc_t, ilo, frac)

    return ece.reshape(1)


# speedup vs baseline: 1.6803x; 1.1119x over previous
import jax
import jax.numpy as jnp
from jax import lax
from jax.experimental import pallas as pl
from jax.experimental.pallas import tpu as pltpu

_N_BINS = 15
_NB = 100
_MAX_BITS = 0x3F800000


def _pass1(probs_ref, lab_ref, conf_ref, acc_ref):
    x = probs_ref[...]
    r = x.shape[0]
    col = lax.broadcasted_iota(jnp.int32, x.shape, 1)
    colf = col.astype(jnp.float32)
    x = jnp.where(col == 1, -9999.0, x)
    conf = jnp.max(x, axis=1, keepdims=True)
    first = jnp.min(jnp.where(x == conf, colf, 128.0), axis=1, keepdims=True)
    conf_row = jnp.transpose(conf)
    first_row = jnp.transpose(first)
    lab = lab_ref[...].reshape(1, r)
    acc_row = (first_row == lab).astype(jnp.float32)
    conf_ref[...] = conf_row.reshape(1, 1, r)
    acc_ref[...] = acc_row.reshape(1, 1, r)


def _pass2(conf_ref, acc_ref, ranks_ref, frac_ref, out_ref, bits_ref):
    nb, _, r = conf_ref.shape
    n = nb * r
    bits_ref[...] = lax.bitcast_convert_type(
        conf_ref[...].reshape(nb, r), jnp.int32)

    ranks = [ranks_ref[j] for j in range(16)]

    def body(_, st):
        los, his = st
        mids = [lo + (hi - lo) // 2 for lo, hi in zip(los, his)]
        b = bits_ref[...]
        new_lo, new_hi = [], []
        for j in range(16):
            c = jnp.sum((b <= mids[j]).astype(jnp.int32))
            take = c >= ranks[j] + 1
            new_lo.append(jnp.where(take, los[j], mids[j] + 1))
            new_hi.append(jnp.where(take, mids[j], his[j]))
        return (tuple(new_lo), tuple(new_hi))

    init = (tuple(jnp.int32(0) for _ in range(16)),
            tuple(jnp.int32(_MAX_BITS - 1) for _ in range(16)))
    los, _ = lax.fori_loop(0, 30, body, init)

    b = bits_ref[...]
    bounds = []
    for j in range(16):
        lo = los[j]
        cnt_le = jnp.sum((b <= lo).astype(jnp.int32))
        nxt = jnp.min(jnp.where(b > lo, b, jnp.int32(0x7F7FFFFF)))
        hi_bits = jnp.where(cnt_le >= ranks[j] + 2, lo, nxt)
        v_lo = lax.bitcast_convert_type(lo, jnp.float32)
        v_hi = lax.bitcast_convert_type(hi_bits, jnp.float32)
        f = frac_ref[j]
        bounds.append(v_lo + f * (v_hi - v_lo))

    ece = jnp.float32(0.0)
    conf = conf_ref[...].reshape(nb, r)
    acc = acc_ref[...].reshape(nb, r)
    for i in range(_N_BINS):
        inb = (conf > bounds[i]) & (conf <= bounds[i + 1])
        cnt = jnp.sum(jnp.where(inb, 1.0, 0.0))
        sc = jnp.sum(jnp.where(inb, conf, 0.0))
        sa = jnp.sum(jnp.where(inb, acc, 0.0))
        safe = jnp.maximum(cnt, 1.0)
        contrib = jnp.abs(sc / safe - sa / safe) * (cnt / n)
        ece = ece + jnp.where(cnt > 0, contrib, 0.0)
    out_ref[0, 0] = ece


def kernel(probs, labels):
    n, c = probs.shape
    r = n // _NB
    labels3 = labels.astype(jnp.int32).astype(jnp.float32).reshape(_NB, 1, r)

    conf_t, acc_t = pl.pallas_call(
        _pass1,
        grid=(_NB,),
        in_specs=[
            pl.BlockSpec((r, c), lambda i: (i, 0)),
            pl.BlockSpec((1, 1, r), lambda i: (i, 0, 0)),
        ],
        out_specs=[
            pl.BlockSpec((1, 1, r), lambda i: (i, 0, 0)),
            pl.BlockSpec((1, 1, r), lambda i: (i, 0, 0)),
        ],
        out_shape=[
            jax.ShapeDtypeStruct((_NB, 1, r), jnp.float32),
            jax.ShapeDtypeStruct((_NB, 1, r), jnp.float32),
        ],
    )(probs, labels3)

    xq = jnp.linspace(0.0, float(n), _N_BINS + 1)
    ilo = jnp.clip(jnp.floor(xq), 0, n - 1).astype(jnp.int32)
    frac = jnp.clip(xq - ilo.astype(jnp.float32), 0.0, 1.0)
    frac = jnp.where(ilo >= n - 1, 0.0, frac).astype(jnp.float32)

    ece = pl.pallas_call(
        _pass2,
        in_specs=[
            pl.BlockSpec((_NB, 1, r), lambda: (0, 0, 0)),
            pl.BlockSpec((_NB, 1, r), lambda: (0, 0, 0)),
            pl.BlockSpec(memory_space=pltpu.SMEM),
            pl.BlockSpec(memory_space=pltpu.SMEM),
        ],
        out_specs=pl.BlockSpec(memory_space=pltpu.SMEM),
        out_shape=jax.ShapeDtypeStruct((1, 1), jnp.float32),
        scratch_shapes=[pltpu.VMEM((_NB, r), jnp.int32)],
    )(conf_t, acc_t, ilo, frac)

    return ece.reshape(1)
